# Initial kernel scaffold; baseline (speedup 1.0000x reference)
import jax
import jax.numpy as jnp
from jax.experimental import pallas as pl


def kernel(fnode, fmess, agraph, bgraph, mask, W_z, b_z, W_r, U_r, b_ur, W_h, b_h, W_o, b_o):
    raise NotImplementedError("write your pallas kernel here")


# speedup vs baseline: 3.1065x; 3.1065x over previous
import functools

import jax
import jax.numpy as jnp
from jax import lax
from jax.experimental import pallas as pl
from jax.experimental.pallas import tpu as pltpu
from jax.experimental.pallas import tpu_sc as plsc

H = 128
NB = 4
NW = 32
GROW = 128


def _pick_b0(ngroups):
    for b0 in (5, 4, 2, 1):
        if ngroups % b0 == 0:
            return b0
    return 1


def _sc_gather(table, idxg):
    G = idxg.shape[0]
    g_pw = G // NW
    assert G % NW == 0
    mesh = plsc.VectorSubcoreMesh(core_axis_name="c", subcore_axis_name="s")

    @functools.partial(
        pl.kernel,
        out_type=jax.ShapeDtypeStruct((G, GROW, H), jnp.float32),
        mesh=mesh,
        scratch_types=[
            pltpu.VMEM((GROW,), jnp.int32),
            pltpu.VMEM((GROW, H), jnp.float32),
            pltpu.SemaphoreType.DMA,
        ],
    )
    def k(table_hbm, idx_hbm, out_hbm, idx_v, rows_v, sem):
        wid = lax.axis_index("s") * 2 + lax.axis_index("c")

        def body(i, _):
            g = wid * g_pw + i
            pltpu.sync_copy(idx_hbm.at[g], idx_v)
            pltpu.async_copy(table_hbm.at[idx_v], rows_v, sem).wait()
            pltpu.sync_copy(rows_v, out_hbm.at[g])
            return ()

        lax.fori_loop(0, g_pw, body, (), unroll=False)

    return k(table, idxg)


def _tc_depth0(fmess, Wc, bc):
    E, FD = fmess.shape
    BE = 640 if E % 640 == 0 else 512
    nblk = E // BE

    def body(fm, wc, b, out):
        x = jnp.dot(fm[...], wc[...], preferred_element_type=jnp.float32) + b[...]
        fz = x[:, :H]
        fh = x[:, H:]
        h1 = jax.nn.sigmoid(fz) * jnp.tanh(fh)
        rows = lax.broadcasted_iota(jnp.int32, (BE, 1), 0) + pl.program_id(0) * BE
        out[...] = jnp.where(rows == 0, 0.0, h1)

    return pl.pallas_call(
        body,
        grid=(nblk,),
        in_specs=[
            pl.BlockSpec((BE, FD), lambda j: (j, 0)),
            pl.BlockSpec((FD, 2 * H), lambda j: (0, 0)),
            pl.BlockSpec((1, 2 * H), lambda j: (0, 0)),
        ],
        out_specs=pl.BlockSpec((BE, H), lambda j: (j, 0)),
        out_shape=jax.ShapeDtypeStruct((E, H), jnp.float32),
    )(fmess, Wc, bc)


def _tc_depth(hn, fmess, Wc, bc, U_r, b_ur, Wz1, Wh1):
    E, FD = fmess.shape
    GK = E // GROW
    B0 = _pick_b0(GK)
    BE = B0 * GROW
    nblk = E // BE

    def body(hn0, hn1, hn2, hn3, fm, wc, b, ur, bur, wz, wh, out):
        x = jnp.dot(fm[...], wc[...], preferred_element_type=jnp.float32) + b[...]
        fz = x[:, :H]
        r1 = x[:, H:2 * H]
        fh = x[:, 2 * H:]
        sum_h = jnp.zeros((BE, H), jnp.float32)
        sum_g = jnp.zeros((BE, H), jnp.float32)
        for ref in (hn0, hn1, hn2, hn3):
            hk = ref[...].reshape(BE, H)
            r2 = jnp.dot(hk, ur[...], preferred_element_type=jnp.float32)
            rk = jax.nn.sigmoid(r1 + r2 + bur[...])
            sum_h = sum_h + hk
            sum_g = sum_g + rk * hk
        z = jax.nn.sigmoid(fz + jnp.dot(sum_h, wz[...], preferred_element_type=jnp.float32))
        pre = jnp.tanh(fh + jnp.dot(sum_g, wh[...], preferred_element_type=jnp.float32))
        newh = (1.0 - z) * sum_h + z * pre
        rows = lax.broadcasted_iota(jnp.int32, (BE, 1), 0) + pl.program_id(0) * BE
        out[...] = jnp.where(rows == 0, 0.0, newh)

    def hn_spec(k):
        return pl.BlockSpec((B0, GROW, H), lambda j, k=k: (k * (GK // B0) + j, 0, 0))

    return pl.pallas_call(
        body,
        grid=(nblk,),
        in_specs=[
            hn_spec(0), hn_spec(1), hn_spec(2), hn_spec(3),
            pl.BlockSpec((BE, FD), lambda j: (j, 0)),
            pl.BlockSpec((FD, 3 * H), lambda j: (0, 0)),
            pl.BlockSpec((1, 3 * H), lambda j: (0, 0)),
            pl.BlockSpec((H, H), lambda j: (0, 0)),
            pl.BlockSpec((1, H), lambda j: (0, 0)),
            pl.BlockSpec((H, H), lambda j: (0, 0)),
            pl.BlockSpec((H, H), lambda j: (0, 0)),
        ],
        out_specs=pl.BlockSpec((BE, H), lambda j: (j, 0)),
        out_shape=jax.ShapeDtypeStruct((E, H), jnp.float32),
    )(hn, hn, hn, hn, fmess, Wc, bc, U_r, b_ur, Wz1, Wh1)


def _tc_readout(an, fnode_p, Wo0, Wo1, bo, mask_p):
    NP, FD = fnode_p.shape
    GKN = NP // GROW
    B0 = _pick_b0(GKN)
    BN = B0 * GROW
    nblk = NP // BN

    def body(a0, a1, a2, a3, fn, w0, w1, b, m, out):
        nei = (a0[...] + a1[...] + a2[...] + a3[...]).reshape(BN, H)
        acc = jnp.dot(fn[...], w0[...], preferred_element_type=jnp.float32)
        acc = acc + jnp.dot(nei, w1[...], preferred_element_type=jnp.float32)
        out[...] = jax.nn.relu(acc + b[...]) * m[...]

    def an_spec(k):
        return pl.BlockSpec((B0, GROW, H), lambda j, k=k: (k * (GKN // B0) + j, 0, 0))

    return pl.pallas_call(
        body,
        grid=(nblk,),
        in_specs=[
            an_spec(0), an_spec(1), an_spec(2), an_spec(3),
            pl.BlockSpec((BN, FD), lambda j: (j, 0)),
            pl.BlockSpec((FD, H), lambda j: (0, 0)),
            pl.BlockSpec((H, H), lambda j: (0, 0)),
            pl.BlockSpec((1, H), lambda j: (0, 0)),
            pl.BlockSpec((BN, 1), lambda j: (j, 0)),
        ],
        out_specs=pl.BlockSpec((BN, H), lambda j: (j, 0)),
        out_shape=jax.ShapeDtypeStruct((NP, H), jnp.float32),
    )(an, an, an, an, fnode_p, Wo0, Wo1, bo, mask_p)


def kernel(fnode, fmess, agraph, bgraph, mask, W_z, b_z, W_r, U_r, b_ur, W_h, b_h, W_o, b_o):
    N, NFD = fnode.shape
    E, FD = fmess.shape

    Wz0, Wz1 = W_z[:FD], W_z[FD:]
    Wh0, Wh1 = W_h[:FD], W_h[FD:]
    Wo0, Wo1 = W_o[:NFD], W_o[NFD:]
    Wc0 = jnp.concatenate([Wz0, Wh0], axis=1)
    bc0 = jnp.concatenate([b_z, b_h])[None, :]
    Wc = jnp.concatenate([Wz0, W_r, Wh0], axis=1)
    bc = jnp.concatenate([b_z, jnp.zeros_like(b_z), b_h])[None, :]
    bur = b_ur[None, :]
    bo = b_o[None, :]

    RB = NB * E
    RBp = ((RB + NW * GROW - 1) // (NW * GROW)) * (NW * GROW)
    bg_idx = jnp.concatenate(
        [bgraph.T.reshape(-1), jnp.zeros((RBp - RB,), jnp.int32)]
    ).reshape(-1, GROW)

    NP = ((N + 1023) // 1024) * 1024
    ag_idx = jnp.pad(agraph.T, ((0, 0), (0, NP - N))).reshape(-1, GROW)
    fnode_p = jnp.pad(fnode, ((0, NP - N), (0, 0)))
    mask_p = jnp.pad(mask, ((0, NP - N), (0, 0)))

    h = _tc_depth0(fmess, Wc0, bc0)
    for _ in range(2):
        hn = _sc_gather(h, bg_idx)
        h = _tc_depth(hn, fmess, Wc, bc, U_r, bur, Wz1, Wh1)
    an = _sc_gather(h, ag_idx)
    out = _tc_readout(an, fnode_p, Wo0, Wo1, bo, mask_p)
    return out[:N]


# speedup vs baseline: 3.3507x; 1.0786x over previous
import functools

import jax
import jax.numpy as jnp
from jax import lax
from jax.experimental import pallas as pl
from jax.experimental.pallas import tpu as pltpu
from jax.experimental.pallas import tpu_sc as plsc

H = 128
NB = 4
NW = 32
GROW = 128


def _pick_b0(ngroups):
    for b0 in (5, 4, 2, 1):
        if ngroups % b0 == 0:
            return b0
    return 1


def _sc_gather(table, idxg):
    G = idxg.shape[0]
    g_pw = G // NW
    assert G % NW == 0
    NBUF = 4 if g_pw % 4 == 0 else (2 if g_pw % 2 == 0 else 1)
    mesh = plsc.VectorSubcoreMesh(core_axis_name="c", subcore_axis_name="s")

    @functools.partial(
        pl.kernel,
        out_type=jax.ShapeDtypeStruct((G, GROW, H), jnp.float32),
        mesh=mesh,
        scratch_types=[
            pltpu.VMEM((g_pw * GROW,), jnp.int32),
            pltpu.VMEM((NBUF * GROW, H), jnp.float32),
            pltpu.SemaphoreType.DMA,
            [pltpu.SemaphoreType.DMA] * NBUF,
            [pltpu.SemaphoreType.DMA] * NBUF,
        ],
    )
    def k(table_hbm, idx_hbm, out_hbm, idx_v, rows_v, isem, gsems, wsems):
        wid = lax.axis_index("s") * 2 + lax.axis_index("c")
        base = wid * g_pw

        def idx_at(g):
            return idx_v.at[pl.ds(pl.multiple_of(g * GROW, GROW), GROW)]

        def buf_at(b):
            return rows_v.at[pl.ds(b * GROW, GROW)]

        pltpu.async_copy(
            idx_hbm.at[pl.ds(base * GROW, g_pw * GROW)], idx_v, isem
        ).wait()

        for b in range(NBUF):
            pltpu.async_copy(table_hbm.at[idx_at(b)], buf_at(b), gsems[b])

        def body(i, _):
            for b in range(NBUF):
                g = i * NBUF + b
                pltpu.make_async_copy(table_hbm.at[idx_at(g)], buf_at(b), gsems[b]).wait()
                wr = pltpu.async_copy(buf_at(b), out_hbm.at[base + g], wsems[b])
                @pl.when(i < g_pw // NBUF - 1)
                def _():
                    wr.wait()
                    pltpu.async_copy(table_hbm.at[idx_at(g + NBUF)], buf_at(b), gsems[b])
            return ()

        lax.fori_loop(0, g_pw // NBUF, body, (), unroll=False)
        for b in range(NBUF):
            pltpu.make_async_copy(buf_at(b), out_hbm.at[base], wsems[b]).wait()

    return k(table, idxg.reshape(-1))


def _tc_depth0(fmess, Wc, bc):
    E, FD = fmess.shape
    BE = 640 if E % 640 == 0 else 512
    nblk = E // BE

    def body(fm, wc, b, out):
        x = jnp.dot(fm[...], wc[...], preferred_element_type=jnp.float32) + b[...]
        fz = x[:, :H]
        fh = x[:, H:]
        h1 = jax.nn.sigmoid(fz) * jnp.tanh(fh)
        rows = lax.broadcasted_iota(jnp.int32, (BE, 1), 0) + pl.program_id(0) * BE
        out[...] = jnp.where(rows == 0, 0.0, h1)

    return pl.pallas_call(
        body,
        grid=(nblk,),
        in_specs=[
            pl.BlockSpec((BE, FD), lambda j: (j, 0)),
            pl.BlockSpec((FD, 2 * H), lambda j: (0, 0)),
            pl.BlockSpec((1, 2 * H), lambda j: (0, 0)),
        ],
        out_specs=pl.BlockSpec((BE, H), lambda j: (j, 0)),
        out_shape=jax.ShapeDtypeStruct((E, H), jnp.float32),
    )(fmess, Wc, bc)


def _tc_depth(hn, fmess, Wc, bc, U_r, b_ur, Wz1, Wh1):
    E, FD = fmess.shape
    GK = E // GROW
    B0 = _pick_b0(GK)
    BE = B0 * GROW
    nblk = E // BE

    def body(hn0, hn1, hn2, hn3, fm, wc, b, ur, bur, wz, wh, out):
        x = jnp.dot(fm[...], wc[...], preferred_element_type=jnp.float32) + b[...]
        fz = x[:, :H]
        r1 = x[:, H:2 * H]
        fh = x[:, 2 * H:]
        sum_h = jnp.zeros((BE, H), jnp.float32)
        sum_g = jnp.zeros((BE, H), jnp.float32)
        for ref in (hn0, hn1, hn2, hn3):
            hk = ref[...].reshape(BE, H)
            r2 = jnp.dot(hk, ur[...], preferred_element_type=jnp.float32)
            rk = jax.nn.sigmoid(r1 + r2 + bur[...])
            sum_h = sum_h + hk
            sum_g = sum_g + rk * hk
        z = jax.nn.sigmoid(fz + jnp.dot(sum_h, wz[...], preferred_element_type=jnp.float32))
        pre = jnp.tanh(fh + jnp.dot(sum_g, wh[...], preferred_element_type=jnp.float32))
        newh = (1.0 - z) * sum_h + z * pre
        rows = lax.broadcasted_iota(jnp.int32, (BE, 1), 0) + pl.program_id(0) * BE
        out[...] = jnp.where(rows == 0, 0.0, newh)

    def hn_spec(k):
        return pl.BlockSpec((B0, GROW, H), lambda j, k=k: (k * (GK // B0) + j, 0, 0))

    return pl.pallas_call(
        body,
        grid=(nblk,),
        in_specs=[
            hn_spec(0), hn_spec(1), hn_spec(2), hn_spec(3),
            pl.BlockSpec((BE, FD), lambda j: (j, 0)),
            pl.BlockSpec((FD, 3 * H), lambda j: (0, 0)),
            pl.BlockSpec((1, 3 * H), lambda j: (0, 0)),
            pl.BlockSpec((H, H), lambda j: (0, 0)),
            pl.BlockSpec((1, H), lambda j: (0, 0)),
            pl.BlockSpec((H, H), lambda j: (0, 0)),
            pl.BlockSpec((H, H), lambda j: (0, 0)),
        ],
        out_specs=pl.BlockSpec((BE, H), lambda j: (j, 0)),
        out_shape=jax.ShapeDtypeStruct((E, H), jnp.float32),
    )(hn, hn, hn, hn, fmess, Wc, bc, U_r, b_ur, Wz1, Wh1)


def _tc_readout(an, fnode_p, Wo0, Wo1, bo, mask_p):
    NP, FD = fnode_p.shape
    GKN = NP // GROW
    B0 = _pick_b0(GKN)
    BN = B0 * GROW
    nblk = NP // BN

    def body(a0, a1, a2, a3, fn, w0, w1, b, m, out):
        nei = (a0[...] + a1[...] + a2[...] + a3[...]).reshape(BN, H)
        acc = jnp.dot(fn[...], w0[...], preferred_element_type=jnp.float32)
        acc = acc + jnp.dot(nei, w1[...], preferred_element_type=jnp.float32)
        out[...] = jax.nn.relu(acc + b[...]) * m[...]

    def an_spec(k):
        return pl.BlockSpec((B0, GROW, H), lambda j, k=k: (k * (GKN // B0) + j, 0, 0))

    return pl.pallas_call(
        body,
        grid=(nblk,),
        in_specs=[
            an_spec(0), an_spec(1), an_spec(2), an_spec(3),
            pl.BlockSpec((BN, FD), lambda j: (j, 0)),
            pl.BlockSpec((FD, H), lambda j: (0, 0)),
            pl.BlockSpec((H, H), lambda j: (0, 0)),
            pl.BlockSpec((1, H), lambda j: (0, 0)),
            pl.BlockSpec((BN, 1), lambda j: (j, 0)),
        ],
        out_specs=pl.BlockSpec((BN, H), lambda j: (j, 0)),
        out_shape=jax.ShapeDtypeStruct((NP, H), jnp.float32),
    )(an, an, an, an, fnode_p, Wo0, Wo1, bo, mask_p)


def kernel(fnode, fmess, agraph, bgraph, mask, W_z, b_z, W_r, U_r, b_ur, W_h, b_h, W_o, b_o):
    N, NFD = fnode.shape
    E, FD = fmess.shape

    Wz0, Wz1 = W_z[:FD], W_z[FD:]
    Wh0, Wh1 = W_h[:FD], W_h[FD:]
    Wo0, Wo1 = W_o[:NFD], W_o[NFD:]
    Wc0 = jnp.concatenate([Wz0, Wh0], axis=1)
    bc0 = jnp.concatenate([b_z, b_h])[None, :]
    Wc = jnp.concatenate([Wz0, W_r, Wh0], axis=1)
    bc = jnp.concatenate([b_z, jnp.zeros_like(b_z), b_h])[None, :]
    bur = b_ur[None, :]
    bo = b_o[None, :]

    RB = NB * E
    RBp = ((RB + NW * GROW - 1) // (NW * GROW)) * (NW * GROW)
    bg_idx = jnp.concatenate(
        [bgraph.T.reshape(-1), jnp.zeros((RBp - RB,), jnp.int32)]
    ).reshape(-1, GROW)

    NP = ((N + 1023) // 1024) * 1024
    ag_idx = jnp.pad(agraph.T, ((0, 0), (0, NP - N))).reshape(-1, GROW)
    fnode_p = jnp.pad(fnode, ((0, NP - N), (0, 0)))
    mask_p = jnp.pad(mask, ((0, NP - N), (0, 0)))

    h = _tc_depth0(fmess, Wc0, bc0)
    for _ in range(2):
        hn = _sc_gather(h, bg_idx)
        h = _tc_depth(hn, fmess, Wc, bc, U_r, bur, Wz1, Wh1)
    an = _sc_gather(h, ag_idx)
    out = _tc_readout(an, fnode_p, Wo0, Wo1, bo, mask_p)
    return out[:N]
